# baseline (speedup 1.0000x reference)
import functools

import jax
import jax.numpy as jnp
from jax import lax
from jax.experimental import pallas as pl
from jax.experimental.pallas import tpu as pltpu
from jax.experimental.pallas import tpu_sc as plsc

VOCAB = 1000000
EMBED_DIM = 64

NUM_CORES = 2
NUM_SUBCORES = 16
NUM_WORKERS = NUM_CORES * NUM_SUBCORES

CHUNK = 128
NBUF = 4


def _make_sc_gather(total_rows: int):
    assert total_rows % (NUM_WORKERS * CHUNK) == 0
    chunks_per_worker = total_rows // (NUM_WORKERS * CHUNK)
    assert chunks_per_worker % NBUF == 0 and chunks_per_worker > NBUF
    n_main = chunks_per_worker - NBUF

    mesh = plsc.VectorSubcoreMesh(core_axis_name="c", subcore_axis_name="s")

    @functools.partial(
        pl.kernel,
        out_type=jax.ShapeDtypeStruct((total_rows // CHUNK, CHUNK, EMBED_DIM),
                                      jnp.float32),
        mesh=mesh,
        scratch_types=[
            pltpu.VMEM((chunks_per_worker, CHUNK), jnp.int32),
            pltpu.VMEM((NBUF, CHUNK, EMBED_DIM), jnp.float32),
            [pltpu.SemaphoreType.DMA] * NBUF,
            [pltpu.SemaphoreType.DMA] * NBUF,
        ],
        compiler_params=pltpu.CompilerParams(use_tc_tiling_on_sc=False),
    )
    def gather_kernel(idx_hbm, table_hbm, out_hbm, idx_v, bufs, gsems, osems):
        wid = lax.axis_index("s") * NUM_CORES + lax.axis_index("c")
        chunk_base = wid * chunks_per_worker

        pltpu.sync_copy(idx_hbm.at[pl.ds(chunk_base, chunks_per_worker)], idx_v)

        def start_gather(g, b):
            pltpu.async_copy(table_hbm.at[idx_v.at[g]], bufs.at[b], gsems[b])

        def start_out(g, b):
            pltpu.async_copy(bufs.at[b], out_hbm.at[chunk_base + g], osems[b])

        def wait_gather(g, b):
            pltpu.make_async_copy(table_hbm.at[idx_v.at[g]], bufs.at[b],
                                  gsems[b]).wait()

        def wait_out(g, b):
            pltpu.make_async_copy(bufs.at[b], out_hbm.at[chunk_base + g],
                                  osems[b]).wait()

        for b in range(NBUF):
            start_gather(b, b)

        def body(j):
            for b in range(NBUF):
                g = j + b
                wait_gather(g, b)
                start_out(g, b)
                wait_out(g, b)
                start_gather(g + NBUF, b)

        pl.loop(0, n_main, step=NBUF)(body)

        for b in range(NBUF):
            g = n_main + b
            wait_gather(g, b)
            start_out(g, b)
            wait_out(g, b)

    return gather_kernel


def kernel(indices, table):
    b, s = indices.shape
    total = b * s
    idx2d = indices.astype(jnp.int32).reshape(total // CHUNK, CHUNK)
    out = _make_sc_gather(total)(idx2d, table)
    return out.reshape(b, s, EMBED_DIM)
